# baseline (speedup 1.0000x reference)
import functools

import jax
import jax.numpy as jnp
from jax import lax
from jax.experimental import pallas as pl
from jax.experimental.pallas import tpu as pltpu
from jax.experimental.pallas import tpu_sc as plsc

B = 16384
EMB = 16
NC = 2
NS = 16
L = 16
NW = NC * NS
BPW = B // NW
NCH = 8
CH = BPW // NCH
VOCAB = 1000000
EPG = 128 // EMB
LIN_ROWS = -(-VOCAB // 128)


@functools.partial(
    pl.kernel,
    mesh=plsc.VectorSubcoreMesh(core_axis_name="c", subcore_axis_name="s"),
    compiler_params=pltpu.CompilerParams(needs_layout_passes=False),
    out_type=[
        jax.ShapeDtypeStruct((NW, BPW), jnp.float32),
        jax.ShapeDtypeStruct((NW, L), jnp.float32),
    ],
    scratch_types=[
        pltpu.VMEM((BPW,), jnp.int32),
        pltpu.VMEM((BPW,), jnp.int32),
        pltpu.VMEM((BPW,), jnp.int32),
        pltpu.VMEM((BPW,), jnp.int32),
        pltpu.VMEM((BPW,), jnp.int32),
        pltpu.VMEM((BPW,), jnp.int32),
        pltpu.VMEM((2, CH, 128), jnp.float32),
        pltpu.VMEM((2, CH, 128), jnp.float32),
        pltpu.VMEM((2, CH, 128), jnp.float32),
        pltpu.VMEM((2, CH, 128), jnp.float32),
        pltpu.VMEM((BPW,), jnp.float32),
        pltpu.VMEM((L,), jnp.float32),
        pltpu.SemaphoreType.DMA,
        pltpu.SemaphoreType.DMA,
    ],
)
def _fm_sc(users_hbm, items_hbm, uemb_hbm, iemb_hbm, ulin_hbm, ilin_hbm,
           out_hbm, aux_hbm,
           uidx_v, iidx_v, ueg_v, ieg_v, ulg_v, ilg_v,
           ue_v, ie_v, ul_v, il_v, out_v, acc_v, sem_a, sem_b):
    wid = lax.axis_index("s") * NC + lax.axis_index("c")

    pltpu.sync_copy(users_hbm.at[wid], uidx_v)
    pltpu.sync_copy(items_hbm.at[wid], iidx_v)

    for i in range(BPW // L):
        s = pl.ds(i * L, L)
        u = uidx_v[s]
        t = iidx_v[s]
        ueg_v[s] = u >> 3
        ieg_v[s] = t >> 3
        ulg_v[s] = u >> 7
        ilg_v[s] = t >> 7

    def start(ch):
        p = ch % 2
        sem = sem_a if p == 0 else sem_b
        rows = pl.ds(ch * CH, CH)
        return [
            pltpu.async_copy(uemb_hbm.at[ueg_v.at[rows]], ue_v.at[p], sem),
            pltpu.async_copy(iemb_hbm.at[ieg_v.at[rows]], ie_v.at[p], sem),
            pltpu.async_copy(ulin_hbm.at[ulg_v.at[rows]], ul_v.at[p], sem),
            pltpu.async_copy(ilin_hbm.at[ilg_v.at[rows]], il_v.at[p], sem),
        ]

    acc = jnp.zeros((L,), jnp.float32)
    inflight = start(0)
    for ch in range(NCH):
        for cp in inflight:
            cp.wait()
        if ch + 1 < NCH:
            inflight = start(ch + 1)
        p = ch % 2
        pp = jnp.full((L,), p, jnp.int32)
        for g in range(CH // L):
            rloc = g * L + lax.iota(jnp.int32, L)
            s = pl.ds(ch * CH + g * L, L)
            uid = uidx_v[s]
            iid = iidx_v[s]
            uoff = (uid & (EPG - 1)) * EMB
            ioff = (iid & (EPG - 1)) * EMB
            fm = jnp.zeros((L,), jnp.float32)
            for d in range(EMB):
                uc = plsc.load_gather(ue_v, [pp, rloc, uoff + d])
                ic = plsc.load_gather(ie_v, [pp, rloc, ioff + d])
                fm = fm + uc * ic
            ul = plsc.load_gather(ul_v, [pp, rloc, uid & 127])
            il = plsc.load_gather(il_v, [pp, rloc, iid & 127])
            x = ul + il + fm
            sig = 1.0 / (1.0 + jnp.exp(-x))
            out_v[s] = sig
            acc = acc + fm * fm

    acc_v[...] = acc
    pltpu.sync_copy(out_v, out_hbm.at[wid])
    pltpu.sync_copy(acc_v, aux_hbm.at[wid])


def kernel(users, items, user_emb, item_emb, user_lin, item_lin):
    u = users.reshape(NW, BPW).astype(jnp.int32)
    i = items.reshape(NW, BPW).astype(jnp.int32)
    uemb = user_emb.reshape(VOCAB // EPG, 128)
    iemb = item_emb.reshape(VOCAB // EPG, 128)
    pad = LIN_ROWS * 128 - VOCAB
    ulin = jnp.pad(user_lin.reshape(-1), (0, pad)).reshape(LIN_ROWS, 128)
    ilin = jnp.pad(item_lin.reshape(-1), (0, pad)).reshape(LIN_ROWS, 128)
    sig, parts = _fm_sc(u, i, uemb, iemb, ulin, ilin)
    aux = 0.1 * (jnp.sum(parts) / B)
    return (sig.reshape(B, 1), aux)
